# Initial kernel scaffold; baseline (speedup 1.0000x reference)
import jax
import jax.numpy as jnp
from jax.experimental import pallas as pl


def kernel(domain_batch, words, ext_words, tags, domains, word_lens_encoder, char_idxs_encoder, W_word, W_ext, W_domain, char_table, W_ih_f, W_hh_f, b_ih_f, b_hh_f, W_ih_b, W_hh_b, b_ih_b, b_hh_b):
    raise NotImplementedError("write your pallas kernel here")


# speedup vs baseline: 3.9907x; 3.9907x over previous
import functools

import jax
import jax.numpy as jnp
from jax import lax
from jax.experimental import pallas as pl
from jax.experimental.pallas import tpu as pltpu
from jax.experimental.pallas import tpu_sc as plsc

_B, _S, _LC = 64, 50, 16
_N = _B * _S
_WD, _CD, _CH, _DD = 128, 200, 64, 32
_CDP = 256
_DDP = 128
_G = 4 * _CH
_OUT = _WD + 2 * _CH + _DD

_NC, _NS = 2, 16
_NW = _NC * _NS
_NPAD = 4096
_WB = _NPAD // _NW
_CCH = 80
_CNCH = (_N * _LC) // (_NW * _CCH)

_BLK = 128
_NB = _N // _BLK


def _sc_body(words_ref, ext_ref, cidx_ref, dom_ref, domb_ref,
             wword_ref, wext_ref, ctab_ref, wdom_ref,
             xw_ref, xe_ref, ce_ref, xd_ref, db_ref,
             idx_v, rows_v, cidx_v, crows_v, drows_v, dbidx_v, dbrows_v, sem):
    wid = lax.axis_index("s") * _NC + lax.axis_index("c")
    base = wid * _WB

    pltpu.sync_copy(words_ref.at[pl.ds(base, _WB)], idx_v)
    pltpu.async_copy(wword_ref.at[idx_v], rows_v, sem).wait()
    pltpu.sync_copy(rows_v, xw_ref.at[pl.ds(base, _WB)])

    pltpu.sync_copy(ext_ref.at[pl.ds(base, _WB)], idx_v)
    pltpu.async_copy(wext_ref.at[idx_v], rows_v, sem).wait()
    pltpu.sync_copy(rows_v, xe_ref.at[pl.ds(base, _WB)])

    pltpu.sync_copy(dom_ref.at[pl.ds(base, _WB)], idx_v)
    pltpu.async_copy(wdom_ref.at[idx_v], drows_v, sem).wait()
    pltpu.sync_copy(drows_v, xd_ref.at[pl.ds(base, _WB)])

    cbase = wid * (_CNCH * _CCH)
    for k in range(_CNCH):
        pltpu.sync_copy(cidx_ref.at[wid * _CNCH + k], cidx_v)
        pltpu.async_copy(ctab_ref.at[cidx_v], crows_v, sem).wait()
        pltpu.sync_copy(crows_v, ce_ref.at[pl.ds(cbase + k * _CCH, _CCH)])

    @pl.when(wid == 0)
    def _():
        pltpu.sync_copy(domb_ref, dbidx_v)
        pltpu.async_copy(wdom_ref.at[dbidx_v], dbrows_v, sem).wait()
        pltpu.sync_copy(dbrows_v, db_ref)


@functools.cache
def _sc_gather():
    return pl.kernel(
        _sc_body,
        out_type=(
            jax.ShapeDtypeStruct((_NPAD, _WD), jnp.float32),
            jax.ShapeDtypeStruct((_NPAD, _WD), jnp.float32),
            jax.ShapeDtypeStruct((_N * _LC, _CDP), jnp.float32),
            jax.ShapeDtypeStruct((_NPAD, _DDP), jnp.float32),
            jax.ShapeDtypeStruct((_B, _DDP), jnp.float32),
        ),
        mesh=plsc.VectorSubcoreMesh(core_axis_name="c", subcore_axis_name="s",
                                    num_cores=_NC, num_subcores=_NS),
        scratch_types=[
            pltpu.VMEM((_WB,), jnp.int32),
            pltpu.VMEM((_WB, _WD), jnp.float32),
            pltpu.VMEM((_CCH,), jnp.int32),
            pltpu.VMEM((_CCH, _CDP), jnp.float32),
            pltpu.VMEM((_WB, _DDP), jnp.float32),
            pltpu.VMEM((_B,), jnp.int32),
            pltpu.VMEM((_B, _DDP), jnp.float32),
            pltpu.SemaphoreType.DMA,
        ],
    )


def _lstm_body(emb_ref, lens_ref, xw_ref, xe_ref, xd_ref,
               wf_ref, uf_ref, wb_ref, ub_ref, out_ref, pf_ref, pb_ref):
    emb2 = emb_ref[...].reshape(_LC * _BLK, _CDP)
    pf_ref[...] = jnp.dot(emb2, wf_ref[...],
                          preferred_element_type=jnp.float32).reshape(_LC, _BLK, _G)
    pb_ref[...] = jnp.dot(emb2, wb_ref[...],
                          preferred_element_type=jnp.float32).reshape(_LC, _BLK, _G)
    lens = lens_ref[...]

    def run(p_ref, u_ref, times):
        u = u_ref[...]
        h = jnp.zeros((_BLK, _CH), jnp.float32)
        c = jnp.zeros((_BLK, _CH), jnp.float32)
        for t in times:
            g = p_ref[t] + jnp.dot(h, u, preferred_element_type=jnp.float32)
            i = jax.nn.sigmoid(g[:, 0:_CH])
            f = jax.nn.sigmoid(g[:, _CH:2 * _CH])
            gg = jnp.tanh(g[:, 2 * _CH:3 * _CH])
            o = jax.nn.sigmoid(g[:, 3 * _CH:4 * _CH])
            c_new = f * c + i * gg
            h_new = o * jnp.tanh(c_new)
            m = lens > float(t)
            h = jnp.where(m, h_new, h)
            c = jnp.where(m, c_new, c)
        return h

    hf = run(pf_ref, uf_ref, range(_LC))
    hb = run(pb_ref, ub_ref, range(_LC - 1, -1, -1))
    out_ref[...] = jnp.concatenate(
        [xw_ref[...] + xe_ref[...], hf, hb, xd_ref[:, 0:_DD]], axis=1)


def _tc_call(emb_t, lens_col, xw, xe, xd, wf, uf, wb, ub):
    return pl.pallas_call(
        _lstm_body,
        grid=(_NB,),
        in_specs=[
            pl.BlockSpec((_LC, _BLK, _CDP), lambda i: (0, i, 0)),
            pl.BlockSpec((_BLK, 1), lambda i: (i, 0)),
            pl.BlockSpec((_BLK, _WD), lambda i: (i, 0)),
            pl.BlockSpec((_BLK, _WD), lambda i: (i, 0)),
            pl.BlockSpec((_BLK, _DDP), lambda i: (i, 0)),
            pl.BlockSpec((_CDP, _G), lambda i: (0, 0)),
            pl.BlockSpec((_CH, _G), lambda i: (0, 0)),
            pl.BlockSpec((_CDP, _G), lambda i: (0, 0)),
            pl.BlockSpec((_CH, _G), lambda i: (0, 0)),
        ],
        out_specs=pl.BlockSpec((_BLK, _OUT), lambda i: (i, 0)),
        out_shape=jax.ShapeDtypeStruct((_N, _OUT), jnp.float32),
        scratch_shapes=[
            pltpu.VMEM((_LC, _BLK, _G), jnp.float32),
            pltpu.VMEM((_LC, _BLK, _G), jnp.float32),
        ],
    )(emb_t, lens_col, xw, xe, xd, wf, uf, wb, ub)


def kernel(domain_batch, words, ext_words, tags, domains, word_lens_encoder,
           char_idxs_encoder, W_word, W_ext, W_domain, char_table,
           W_ih_f, W_hh_f, b_ih_f, b_hh_f, W_ih_b, W_hh_b, b_ih_b, b_hh_b):
    pad = _NPAD - _N
    words_p = jnp.pad(words.reshape(-1).astype(jnp.int32), (0, pad))
    ext_p = jnp.pad(ext_words.reshape(-1).astype(jnp.int32), (0, pad))
    dom_p = jnp.pad(domains.reshape(-1).astype(jnp.int32), (0, pad))
    cidx = char_idxs_encoder.astype(jnp.int32).T.reshape(_NW * _CNCH, _CCH)
    ctab = jnp.pad(char_table, ((0, 0), (0, _CDP - _CD))).at[:, _CD].set(1.0)
    wf = jnp.pad(W_ih_f.T, ((0, _CDP - _CD), (0, 0))).at[_CD].set(b_ih_f + b_hh_f)
    wb = jnp.pad(W_ih_b.T, ((0, _CDP - _CD), (0, 0))).at[_CD].set(b_ih_b + b_hh_b)
    uf = W_hh_f.T
    ub = W_hh_b.T
    lens_col = word_lens_encoder.reshape(_N, 1).astype(jnp.float32)

    wdom_p = jnp.pad(W_domain, ((0, 0), (0, _DDP - _DD)))
    xw, xe, ce, xd, db = _sc_gather()(words_p, ext_p, cidx, dom_p,
                                      domain_batch.astype(jnp.int32),
                                      W_word, W_ext, ctab, wdom_p)
    emb_t = ce.reshape(_LC, _N, _CDP)
    x2d = _tc_call(emb_t, lens_col, xw[:_N], xe[:_N], xd[:_N], wf, uf, wb, ub)
    return x2d.reshape(_B, _S, _OUT), db[:, :_DD]


# speedup vs baseline: 5.1717x; 1.2960x over previous
import functools

import jax
import jax.numpy as jnp
from jax import lax
from jax.experimental import pallas as pl
from jax.experimental.pallas import tpu as pltpu
from jax.experimental.pallas import tpu_sc as plsc

_B, _S, _LC = 64, 50, 16
_N = _B * _S
_WD, _CD, _CH, _DD = 128, 200, 64, 32
_CDP = 256
_DDP = 128
_G = 4 * _CH
_OUT = _WD + 2 * _CH + _DD

_NC, _NS = 2, 16
_NW = _NC * _NS
_NPAD = 4096
_WB = _NPAD // _NW
_CCH = 80
_CNCH = (_N * _LC) // (_NW * _CCH)
_NBUF = 3

_BLK = 128
_NB = _N // _BLK


def _sc_body(words_ref, ext_ref, cidx_ref, dom_ref, domb_ref,
             wword_ref, wext_ref, ctab_ref, wdom_ref,
             xw_ref, xe_ref, ce_ref, xd_ref, db_ref,
             widx_v, eidx_v, didx_v, dbidx_v, cidx_v,
             wrows_v, erows_v, drows_v, dbrows_v, cbufs,
             wsem, esem, dsem, dbsem, gsems, osems):
    wid = lax.axis_index("s") * _NC + lax.axis_index("c")
    base = wid * _WB

    pltpu.sync_copy(words_ref.at[pl.ds(base, _WB)], widx_v)
    pltpu.sync_copy(ext_ref.at[pl.ds(base, _WB)], eidx_v)
    pltpu.sync_copy(dom_ref.at[pl.ds(base, _WB)], didx_v)
    pltpu.sync_copy(cidx_ref.at[wid], cidx_v)

    gw = pltpu.async_copy(wword_ref.at[widx_v], wrows_v, wsem)
    ge = pltpu.async_copy(wext_ref.at[eidx_v], erows_v, esem)
    gd = pltpu.async_copy(wdom_ref.at[didx_v], drows_v, dsem)

    gdb = None
    @pl.when(wid < 8)
    def _():
        pltpu.sync_copy(domb_ref.at[pl.ds(wid * 8, 8)], dbidx_v)

    @pl.when(wid < 8)
    def _():
        pltpu.async_copy(wdom_ref.at[dbidx_v], dbrows_v, dbsem).wait()
        pltpu.sync_copy(dbrows_v, db_ref.at[pl.ds(wid * 8, 8)])

    cbase = wid * (_CNCH * _CCH)
    cg = [None] * _CNCH
    cw = [None] * _CNCH
    for k in range(_NBUF):
        cg[k] = pltpu.async_copy(ctab_ref.at[cidx_v.at[k]], cbufs[k], gsems[k])
    for k in range(_CNCH):
        b = k % _NBUF
        cg[k].wait()
        cw[k] = pltpu.async_copy(
            cbufs[b], ce_ref.at[pl.ds(cbase + k * _CCH, _CCH)], osems[b])
        if k + _NBUF < _CNCH:
            cw[k].wait()
            cg[k + _NBUF] = pltpu.async_copy(
                ctab_ref.at[cidx_v.at[k + _NBUF]], cbufs[b], gsems[b])
    for k in range(_CNCH - _NBUF, _CNCH):
        cw[k].wait()

    gw.wait()
    pltpu.sync_copy(wrows_v, xw_ref.at[pl.ds(base, _WB)])
    ge.wait()
    pltpu.sync_copy(erows_v, xe_ref.at[pl.ds(base, _WB)])
    gd.wait()
    pltpu.sync_copy(drows_v, xd_ref.at[pl.ds(base, _WB)])


@functools.cache
def _sc_gather():
    return pl.kernel(
        _sc_body,
        out_type=(
            jax.ShapeDtypeStruct((_NPAD, _WD), jnp.float32),
            jax.ShapeDtypeStruct((_NPAD, _WD), jnp.float32),
            jax.ShapeDtypeStruct((_N * _LC, _CDP), jnp.float32),
            jax.ShapeDtypeStruct((_NPAD, _DDP), jnp.float32),
            jax.ShapeDtypeStruct((_B, _DDP), jnp.float32),
        ),
        mesh=plsc.VectorSubcoreMesh(core_axis_name="c", subcore_axis_name="s",
                                    num_cores=_NC, num_subcores=_NS),
        scratch_types=[
            pltpu.VMEM((_WB,), jnp.int32),
            pltpu.VMEM((_WB,), jnp.int32),
            pltpu.VMEM((_WB,), jnp.int32),
            pltpu.VMEM((8,), jnp.int32),
            pltpu.VMEM((_CNCH, _CCH), jnp.int32),
            pltpu.VMEM((_WB, _WD), jnp.float32),
            pltpu.VMEM((_WB, _WD), jnp.float32),
            pltpu.VMEM((_WB, _DDP), jnp.float32),
            pltpu.VMEM((8, _DDP), jnp.float32),
            [pltpu.VMEM((_CCH, _CDP), jnp.float32) for _ in range(_NBUF)],
            pltpu.SemaphoreType.DMA,
            pltpu.SemaphoreType.DMA,
            pltpu.SemaphoreType.DMA,
            pltpu.SemaphoreType.DMA,
            [pltpu.SemaphoreType.DMA for _ in range(_NBUF)],
            [pltpu.SemaphoreType.DMA for _ in range(_NBUF)],
        ],
    )


def _lstm_body(emb_ref, lens_ref, xw_ref, xe_ref, xd_ref,
               wf_ref, uf_ref, wb_ref, ub_ref, out_ref, pf_ref, pb_ref):
    emb2 = emb_ref[...].reshape(_LC * _BLK, _CDP)
    pf_ref[...] = jnp.dot(emb2, wf_ref[...],
                          preferred_element_type=jnp.float32).reshape(_LC, _BLK, _G)
    pb_ref[...] = jnp.dot(emb2, wb_ref[...],
                          preferred_element_type=jnp.float32).reshape(_LC, _BLK, _G)
    lens = lens_ref[...]

    def run(p_ref, u_ref, times):
        u = u_ref[...]
        h = jnp.zeros((_BLK, _CH), jnp.float32)
        c = jnp.zeros((_BLK, _CH), jnp.float32)
        for t in times:
            g = p_ref[t] + jnp.dot(h, u, preferred_element_type=jnp.float32)
            i = jax.nn.sigmoid(g[:, 0:_CH])
            f = jax.nn.sigmoid(g[:, _CH:2 * _CH])
            gg = jnp.tanh(g[:, 2 * _CH:3 * _CH])
            o = jax.nn.sigmoid(g[:, 3 * _CH:4 * _CH])
            c_new = f * c + i * gg
            h_new = o * jnp.tanh(c_new)
            m = lens > float(t)
            h = jnp.where(m, h_new, h)
            c = jnp.where(m, c_new, c)
        return h

    hf = run(pf_ref, uf_ref, range(_LC))
    hb = run(pb_ref, ub_ref, range(_LC - 1, -1, -1))
    out_ref[...] = jnp.concatenate(
        [xw_ref[...] + xe_ref[...], hf, hb, xd_ref[:, 0:_DD]], axis=1)


def _tc_call(emb_t, lens_col, xw, xe, xd, wf, uf, wb, ub):
    return pl.pallas_call(
        _lstm_body,
        grid=(_NB,),
        in_specs=[
            pl.BlockSpec((_LC, _BLK, _CDP), lambda i: (0, i, 0)),
            pl.BlockSpec((_BLK, 1), lambda i: (i, 0)),
            pl.BlockSpec((_BLK, _WD), lambda i: (i, 0)),
            pl.BlockSpec((_BLK, _WD), lambda i: (i, 0)),
            pl.BlockSpec((_BLK, _DDP), lambda i: (i, 0)),
            pl.BlockSpec((_CDP, _G), lambda i: (0, 0)),
            pl.BlockSpec((_CH, _G), lambda i: (0, 0)),
            pl.BlockSpec((_CDP, _G), lambda i: (0, 0)),
            pl.BlockSpec((_CH, _G), lambda i: (0, 0)),
        ],
        out_specs=pl.BlockSpec((_BLK, _OUT), lambda i: (i, 0)),
        out_shape=jax.ShapeDtypeStruct((_N, _OUT), jnp.float32),
        scratch_shapes=[
            pltpu.VMEM((_LC, _BLK, _G), jnp.float32),
            pltpu.VMEM((_LC, _BLK, _G), jnp.float32),
        ],
    )(emb_t, lens_col, xw, xe, xd, wf, uf, wb, ub)


def kernel(domain_batch, words, ext_words, tags, domains, word_lens_encoder,
           char_idxs_encoder, W_word, W_ext, W_domain, char_table,
           W_ih_f, W_hh_f, b_ih_f, b_hh_f, W_ih_b, W_hh_b, b_ih_b, b_hh_b):
    pad = _NPAD - _N
    words_p = jnp.pad(words.reshape(-1).astype(jnp.int32), (0, pad))
    ext_p = jnp.pad(ext_words.reshape(-1).astype(jnp.int32), (0, pad))
    dom_p = jnp.pad(domains.reshape(-1).astype(jnp.int32), (0, pad))
    cidx = char_idxs_encoder.astype(jnp.int32).T.reshape(_NW, _CNCH, _CCH)
    ctab = jnp.pad(char_table, ((0, 0), (0, _CDP - _CD))).at[:, _CD].set(1.0)
    wf = jnp.pad(W_ih_f.T, ((0, _CDP - _CD), (0, 0))).at[_CD].set(b_ih_f + b_hh_f)
    wb = jnp.pad(W_ih_b.T, ((0, _CDP - _CD), (0, 0))).at[_CD].set(b_ih_b + b_hh_b)
    uf = W_hh_f.T
    ub = W_hh_b.T
    lens_col = word_lens_encoder.reshape(_N, 1).astype(jnp.float32)

    wdom_p = jnp.pad(W_domain, ((0, 0), (0, _DDP - _DD)))
    xw, xe, ce, xd, db = _sc_gather()(words_p, ext_p, cidx, dom_p,
                                      domain_batch.astype(jnp.int32),
                                      W_word, W_ext, ctab, wdom_p)
    emb_t = ce.reshape(_LC, _N, _CDP)
    x2d = _tc_call(emb_t, lens_col, xw[:_N], xe[:_N], xd[:_N], wf, uf, wb, ub)
    return x2d.reshape(_B, _S, _OUT), db[:, :_DD]


# speedup vs baseline: 6.3867x; 1.2349x over previous
import functools

import jax
import jax.numpy as jnp
from jax import lax
from jax.experimental import pallas as pl
from jax.experimental.pallas import tpu as pltpu
from jax.experimental.pallas import tpu_sc as plsc

_B, _S, _LC = 64, 50, 16
_N = _B * _S
_WD, _CD, _CH, _DD = 128, 200, 64, 32
_CDP = 256
_DDP = 128
_G = 4 * _CH
_OUT = _WD + 2 * _CH + _DD

_NC, _NS = 2, 16
_NW = _NC * _NS
_NPAD = 4096
_WB = _NPAD // _NW
_CCH = 80
_CNCH = (_N * _LC) // (_NW * _CCH)
_NBUF = 3

_BLK = 320
_NB = _N // _BLK


def _sc_body(words_ref, ext_ref, cidx_ref, dom_ref, domb_ref,
             wword_ref, wext_ref, ctab_ref, wdom_ref,
             xw_ref, xe_ref, ce_ref, xd_ref, db_ref,
             widx_v, eidx_v, didx_v, dbidx_v, cidx_v,
             wrows_v, erows_v, drows_v, dbrows_v, cbufs,
             wsem, esem, dsem, dbsem, gsems, osems):
    wid = lax.axis_index("s") * _NC + lax.axis_index("c")
    base = wid * _WB

    pltpu.sync_copy(words_ref.at[pl.ds(base, _WB)], widx_v)
    pltpu.sync_copy(ext_ref.at[pl.ds(base, _WB)], eidx_v)
    pltpu.sync_copy(dom_ref.at[pl.ds(base, _WB)], didx_v)
    pltpu.sync_copy(cidx_ref.at[wid], cidx_v)

    gw = pltpu.async_copy(wword_ref.at[widx_v], wrows_v, wsem)
    ge = pltpu.async_copy(wext_ref.at[eidx_v], erows_v, esem)
    gd = pltpu.async_copy(wdom_ref.at[didx_v], drows_v, dsem)

    gdb = None
    @pl.when(wid < 8)
    def _():
        pltpu.sync_copy(domb_ref.at[pl.ds(wid * 8, 8)], dbidx_v)

    @pl.when(wid < 8)
    def _():
        pltpu.async_copy(wdom_ref.at[dbidx_v], dbrows_v, dbsem).wait()
        pltpu.sync_copy(dbrows_v, db_ref.at[pl.ds(wid * 8, 8)])

    cbase = wid * (_CNCH * _CCH)
    cg = [None] * _CNCH
    cw = [None] * _CNCH
    for k in range(_NBUF):
        cg[k] = pltpu.async_copy(ctab_ref.at[cidx_v.at[k]], cbufs[k], gsems[k])
    for k in range(_CNCH):
        b = k % _NBUF
        cg[k].wait()
        cw[k] = pltpu.async_copy(
            cbufs[b], ce_ref.at[pl.ds(cbase + k * _CCH, _CCH)], osems[b])
        if k + _NBUF < _CNCH:
            cw[k].wait()
            cg[k + _NBUF] = pltpu.async_copy(
                ctab_ref.at[cidx_v.at[k + _NBUF]], cbufs[b], gsems[b])
    for k in range(_CNCH - _NBUF, _CNCH):
        cw[k].wait()

    gw.wait()
    pltpu.sync_copy(wrows_v, xw_ref.at[pl.ds(base, _WB)])
    ge.wait()
    pltpu.sync_copy(erows_v, xe_ref.at[pl.ds(base, _WB)])
    gd.wait()
    pltpu.sync_copy(drows_v, xd_ref.at[pl.ds(base, _WB)])


@functools.cache
def _sc_gather():
    return pl.kernel(
        _sc_body,
        out_type=(
            jax.ShapeDtypeStruct((_NPAD, _WD), jnp.float32),
            jax.ShapeDtypeStruct((_NPAD, _WD), jnp.float32),
            jax.ShapeDtypeStruct((_N * _LC, _CDP), jnp.float32),
            jax.ShapeDtypeStruct((_NPAD, _DDP), jnp.float32),
            jax.ShapeDtypeStruct((_B, _DDP), jnp.float32),
        ),
        mesh=plsc.VectorSubcoreMesh(core_axis_name="c", subcore_axis_name="s",
                                    num_cores=_NC, num_subcores=_NS),
        scratch_types=[
            pltpu.VMEM((_WB,), jnp.int32),
            pltpu.VMEM((_WB,), jnp.int32),
            pltpu.VMEM((_WB,), jnp.int32),
            pltpu.VMEM((8,), jnp.int32),
            pltpu.VMEM((_CNCH, _CCH), jnp.int32),
            pltpu.VMEM((_WB, _WD), jnp.float32),
            pltpu.VMEM((_WB, _WD), jnp.float32),
            pltpu.VMEM((_WB, _DDP), jnp.float32),
            pltpu.VMEM((8, _DDP), jnp.float32),
            [pltpu.VMEM((_CCH, _CDP), jnp.float32) for _ in range(_NBUF)],
            pltpu.SemaphoreType.DMA,
            pltpu.SemaphoreType.DMA,
            pltpu.SemaphoreType.DMA,
            pltpu.SemaphoreType.DMA,
            [pltpu.SemaphoreType.DMA for _ in range(_NBUF)],
            [pltpu.SemaphoreType.DMA for _ in range(_NBUF)],
        ],
    )


def _lstm_body(emb_ref, lens_ref, xw_ref, xe_ref, xd_ref,
               wf_ref, uf_ref, bf_ref, wb_ref, ub_ref, bb_ref,
               out_ref, pf_ref, pb_ref):
    emb2 = emb_ref[...].reshape(_LC * _BLK, _CDP)
    pf_ref[...] = (jnp.dot(emb2, wf_ref[...], preferred_element_type=jnp.float32)
                   + bf_ref[...]).reshape(_LC, _BLK, _G)
    pb_ref[...] = (jnp.dot(emb2, wb_ref[...], preferred_element_type=jnp.float32)
                   + bb_ref[...]).reshape(_LC, _BLK, _G)
    lens = lens_ref[...]

    def run(p_ref, u_ref, times):
        u = u_ref[...]
        h = jnp.zeros((_BLK, _CH), jnp.float32)
        c = jnp.zeros((_BLK, _CH), jnp.float32)
        for t in times:
            g = p_ref[t] + jnp.dot(h, u, preferred_element_type=jnp.float32)
            sg = jax.nn.sigmoid(g[:, 0:3 * _CH])
            gg = jnp.tanh(g[:, 3 * _CH:4 * _CH])
            i = sg[:, 0:_CH]
            f = sg[:, _CH:2 * _CH]
            o = sg[:, 2 * _CH:3 * _CH]
            c_new = f * c + i * gg
            h_new = o * jnp.tanh(c_new)
            m = lens > float(t)
            h = jnp.where(m, h_new, h)
            c = jnp.where(m, c_new, c)
        return h

    hf = run(pf_ref, uf_ref, range(_LC))
    hb = run(pb_ref, ub_ref, range(_LC - 1, -1, -1))
    out_ref[...] = jnp.concatenate(
        [xw_ref[...] + xe_ref[...], hf, hb, xd_ref[:, 0:_DD]], axis=1)


def _tc_call(emb_t, lens_col, xw, xe, xd, wf, uf, bf, wb, ub, bb):
    return pl.pallas_call(
        _lstm_body,
        grid=(_NB,),
        in_specs=[
            pl.BlockSpec((_LC, _BLK, _CDP), lambda i: (0, i, 0)),
            pl.BlockSpec((_BLK, 1), lambda i: (i, 0)),
            pl.BlockSpec((_BLK, _WD), lambda i: (i, 0)),
            pl.BlockSpec((_BLK, _WD), lambda i: (i, 0)),
            pl.BlockSpec((_BLK, _DDP), lambda i: (i, 0)),
            pl.BlockSpec((_CDP, _G), lambda i: (0, 0)),
            pl.BlockSpec((_CH, _G), lambda i: (0, 0)),
            pl.BlockSpec((1, _G), lambda i: (0, 0)),
            pl.BlockSpec((_CDP, _G), lambda i: (0, 0)),
            pl.BlockSpec((_CH, _G), lambda i: (0, 0)),
            pl.BlockSpec((1, _G), lambda i: (0, 0)),
        ],
        out_specs=pl.BlockSpec((_BLK, _OUT), lambda i: (i, 0)),
        out_shape=jax.ShapeDtypeStruct((_N, _OUT), jnp.float32),
        scratch_shapes=[
            pltpu.VMEM((_LC, _BLK, _G), jnp.float32),
            pltpu.VMEM((_LC, _BLK, _G), jnp.float32),
        ],
    )(emb_t, lens_col, xw, xe, xd, wf, uf, bf, wb, ub, bb)


def _ifog(w):
    return jnp.concatenate(
        [w[:, 0:2 * _CH], w[:, 3 * _CH:4 * _CH], w[:, 2 * _CH:3 * _CH]], axis=1)


def kernel(domain_batch, words, ext_words, tags, domains, word_lens_encoder,
           char_idxs_encoder, W_word, W_ext, W_domain, char_table,
           W_ih_f, W_hh_f, b_ih_f, b_hh_f, W_ih_b, W_hh_b, b_ih_b, b_hh_b):
    pad = _NPAD - _N
    words_p = jnp.pad(words.reshape(-1).astype(jnp.int32), (0, pad))
    ext_p = jnp.pad(ext_words.reshape(-1).astype(jnp.int32), (0, pad))
    dom_p = jnp.pad(domains.reshape(-1).astype(jnp.int32), (0, pad))
    cidx = char_idxs_encoder.astype(jnp.int32).T.reshape(_NW, _CNCH, _CCH)
    ctab = jnp.pad(char_table, ((0, 0), (0, _CDP - _CD)))
    wf = _ifog(jnp.pad(W_ih_f.T, ((0, _CDP - _CD), (0, 0))))
    wb = _ifog(jnp.pad(W_ih_b.T, ((0, _CDP - _CD), (0, 0))))
    uf = _ifog(W_hh_f.T)
    ub = _ifog(W_hh_b.T)
    bf = _ifog((b_ih_f + b_hh_f).reshape(1, _G))
    bb = _ifog((b_ih_b + b_hh_b).reshape(1, _G))
    lens_col = word_lens_encoder.reshape(_N, 1).astype(jnp.float32)

    wdom_p = jnp.pad(W_domain, ((0, 0), (0, _DDP - _DD)))
    xw, xe, ce, xd, db = _sc_gather()(words_p, ext_p, cidx, dom_p,
                                      domain_batch.astype(jnp.int32),
                                      W_word, W_ext, ctab, wdom_p)
    emb_t = ce.reshape(_LC, _N, _CDP)
    x2d = _tc_call(emb_t, lens_col, xw[:_N], xe[:_N], xd[:_N],
                   wf, uf, bf, wb, ub, bb)
    return x2d.reshape(_B, _S, _OUT), db[:, :_DD]


# speedup vs baseline: 6.6741x; 1.0450x over previous
import functools

import jax
import jax.numpy as jnp
from jax import lax
from jax.experimental import pallas as pl
from jax.experimental.pallas import tpu as pltpu
from jax.experimental.pallas import tpu_sc as plsc

_B, _S, _LC = 64, 50, 16
_CHV = 5000
_N = _B * _S
_WD, _CD, _CH, _DD = 128, 200, 64, 32
_CDP = 256
_DDP = 128
_G = 4 * _CH
_OUT = _WD + 2 * _CH + _DD

_NC, _NS = 2, 16
_NW = _NC * _NS
_NPAD = 4096
_WB = _NPAD // _NW
_CCH = 80
_CNCH = (_N * _LC) // (_NW * _CCH)
_NBUF = 4

_BLK = 320
_NB = _N // _BLK


def _sc_body(words_ref, ext_ref, cidx_ref, dom_ref, domb_ref,
             wword_ref, wext_ref, ctab_ref, wdom_ref,
             xw_ref, xe_ref, ce_ref, xd_ref, db_ref,
             widx_v, eidx_v, didx_v, dbidx_v, cidx_v,
             wrows_v, erows_v, drows_v, dbrows_v, cbufs,
             wsem, esem, dsem, dbsem, gsems, osems):
    wid = lax.axis_index("s") * _NC + lax.axis_index("c")
    base = wid * _WB

    pltpu.sync_copy(words_ref.at[pl.ds(base, _WB)], widx_v)
    pltpu.sync_copy(ext_ref.at[pl.ds(base, _WB)], eidx_v)
    pltpu.sync_copy(dom_ref.at[pl.ds(base, _WB)], didx_v)
    pltpu.sync_copy(cidx_ref.at[wid], cidx_v)

    gw = pltpu.async_copy(wword_ref.at[widx_v], wrows_v, wsem)
    ge = pltpu.async_copy(wext_ref.at[eidx_v], erows_v, esem)
    gd = pltpu.async_copy(wdom_ref.at[didx_v], drows_v, dsem)

    gdb = None
    @pl.when(wid < 8)
    def _():
        pltpu.sync_copy(domb_ref.at[pl.ds(wid * 8, 8)], dbidx_v)

    @pl.when(wid < 8)
    def _():
        pltpu.async_copy(wdom_ref.at[dbidx_v], dbrows_v, dbsem).wait()
        pltpu.sync_copy(dbrows_v, db_ref.at[pl.ds(wid * 8, 8)])

    cbase = wid * (_CNCH * _CCH)
    cg = [None] * _CNCH
    cw = [None] * _CNCH
    for k in range(_NBUF):
        cg[k] = pltpu.async_copy(ctab_ref.at[cidx_v.at[k]], cbufs[k], gsems[k])
    for k in range(_CNCH):
        b = k % _NBUF
        cg[k].wait()
        cw[k] = pltpu.async_copy(
            cbufs[b], ce_ref.at[pl.ds(cbase + k * _CCH, _CCH)], osems[b])
        if k + _NBUF < _CNCH:
            cw[k].wait()
            cg[k + _NBUF] = pltpu.async_copy(
                ctab_ref.at[cidx_v.at[k + _NBUF]], cbufs[b], gsems[b])
    for k in range(_CNCH - _NBUF, _CNCH):
        cw[k].wait()

    gw.wait()
    pltpu.sync_copy(wrows_v, xw_ref.at[pl.ds(base, _WB)])
    ge.wait()
    pltpu.sync_copy(erows_v, xe_ref.at[pl.ds(base, _WB)])
    gd.wait()
    pltpu.sync_copy(drows_v, xd_ref.at[pl.ds(base, _WB)])


@functools.cache
def _sc_gather():
    return pl.kernel(
        _sc_body,
        out_type=(
            jax.ShapeDtypeStruct((_NPAD, _WD), jnp.float32),
            jax.ShapeDtypeStruct((_NPAD, _WD), jnp.float32),
            jax.ShapeDtypeStruct((_N * _LC, 128), jnp.int32),
            jax.ShapeDtypeStruct((_NPAD, _DDP), jnp.float32),
            jax.ShapeDtypeStruct((_B, _DDP), jnp.float32),
        ),
        mesh=plsc.VectorSubcoreMesh(core_axis_name="c", subcore_axis_name="s",
                                    num_cores=_NC, num_subcores=_NS),
        scratch_types=[
            pltpu.VMEM((_WB,), jnp.int32),
            pltpu.VMEM((_WB,), jnp.int32),
            pltpu.VMEM((_WB,), jnp.int32),
            pltpu.VMEM((8,), jnp.int32),
            pltpu.VMEM((_CNCH, _CCH), jnp.int32),
            pltpu.VMEM((_WB, _WD), jnp.float32),
            pltpu.VMEM((_WB, _WD), jnp.float32),
            pltpu.VMEM((_WB, _DDP), jnp.float32),
            pltpu.VMEM((8, _DDP), jnp.float32),
            [pltpu.VMEM((_CCH, 128), jnp.int32) for _ in range(_NBUF)],
            pltpu.SemaphoreType.DMA,
            pltpu.SemaphoreType.DMA,
            pltpu.SemaphoreType.DMA,
            pltpu.SemaphoreType.DMA,
            [pltpu.SemaphoreType.DMA for _ in range(_NBUF)],
            [pltpu.SemaphoreType.DMA for _ in range(_NBUF)],
        ],
    )


def _lstm_body(emb_ref, lens_ref, xw_ref, xe_ref, xd_ref,
               wfe_ref, wfo_ref, uf_ref, bf_ref, wbe_ref, wbo_ref, ub_ref, bb_ref,
               out_ref, pf_ref, pb_ref):
    v = emb_ref[...].reshape(_LC * _BLK, 128)
    e = jax.lax.bitcast_convert_type(v << 16, jnp.float32).astype(jnp.bfloat16)
    o = jax.lax.bitcast_convert_type(v & jnp.int32(-65536),
                                     jnp.float32).astype(jnp.bfloat16)
    pf_ref[...] = (jnp.dot(e, wfe_ref[...], preferred_element_type=jnp.float32)
                   + jnp.dot(o, wfo_ref[...], preferred_element_type=jnp.float32)
                   + bf_ref[...]).reshape(_LC, _BLK, _G)
    pb_ref[...] = (jnp.dot(e, wbe_ref[...], preferred_element_type=jnp.float32)
                   + jnp.dot(o, wbo_ref[...], preferred_element_type=jnp.float32)
                   + bb_ref[...]).reshape(_LC, _BLK, _G)
    lens = lens_ref[...]

    def run(p_ref, u_ref, times):
        u = u_ref[...]
        h = jnp.zeros((_BLK, _CH), jnp.float32)
        c = jnp.zeros((_BLK, _CH), jnp.float32)
        for t in times:
            g = p_ref[t] + jnp.dot(h.astype(jnp.bfloat16), u,
                                   preferred_element_type=jnp.float32)
            sg = jax.nn.sigmoid(g[:, 0:3 * _CH])
            gg = jnp.tanh(g[:, 3 * _CH:4 * _CH])
            i = sg[:, 0:_CH]
            f = sg[:, _CH:2 * _CH]
            o = sg[:, 2 * _CH:3 * _CH]
            c_new = f * c + i * gg
            h_new = o * jnp.tanh(c_new)
            m = lens > float(t)
            h = jnp.where(m, h_new, h)
            c = jnp.where(m, c_new, c)
        return h

    hf = run(pf_ref, uf_ref, range(_LC))
    hb = run(pb_ref, ub_ref, range(_LC - 1, -1, -1))
    out_ref[...] = jnp.concatenate(
        [xw_ref[...] + xe_ref[...], hf, hb, xd_ref[:, 0:_DD]], axis=1)


def _tc_call(emb_t, lens_col, xw, xe, xd, wfe, wfo, uf, bf, wbe, wbo, ub, bb):
    full = lambda s: pl.BlockSpec(s, lambda i: tuple(0 for _ in s))
    return pl.pallas_call(
        _lstm_body,
        grid=(_NB,),
        in_specs=[
            pl.BlockSpec((_LC, _BLK, 128), lambda i: (0, i, 0)),
            pl.BlockSpec((_BLK, 1), lambda i: (i, 0)),
            pl.BlockSpec((_BLK, _WD), lambda i: (i, 0)),
            pl.BlockSpec((_BLK, _WD), lambda i: (i, 0)),
            pl.BlockSpec((_BLK, _DDP), lambda i: (i, 0)),
            full((128, _G)), full((128, _G)), full((_CH, _G)), full((1, _G)),
            full((128, _G)), full((128, _G)), full((_CH, _G)), full((1, _G)),
        ],
        out_specs=pl.BlockSpec((_BLK, _OUT), lambda i: (i, 0)),
        out_shape=jax.ShapeDtypeStruct((_N, _OUT), jnp.float32),
        scratch_shapes=[
            pltpu.VMEM((_LC, _BLK, _G), jnp.float32),
            pltpu.VMEM((_LC, _BLK, _G), jnp.float32),
        ],
    )(emb_t, lens_col, xw, xe, xd, wfe, wfo, uf, bf, wbe, wbo, ub, bb)


def _ifog(w):
    return jnp.concatenate(
        [w[:, 0:2 * _CH], w[:, 3 * _CH:4 * _CH], w[:, 2 * _CH:3 * _CH]], axis=1)


def kernel(domain_batch, words, ext_words, tags, domains, word_lens_encoder,
           char_idxs_encoder, W_word, W_ext, W_domain, char_table,
           W_ih_f, W_hh_f, b_ih_f, b_hh_f, W_ih_b, W_hh_b, b_ih_b, b_hh_b):
    pad = _NPAD - _N
    words_p = jnp.pad(words.reshape(-1).astype(jnp.int32), (0, pad))
    ext_p = jnp.pad(ext_words.reshape(-1).astype(jnp.int32), (0, pad))
    dom_p = jnp.pad(domains.reshape(-1).astype(jnp.int32), (0, pad))
    cidx = char_idxs_encoder.astype(jnp.int32).T.reshape(_NW, _CNCH, _CCH)
    ctab = jax.lax.bitcast_convert_type(
        jnp.pad(char_table, ((0, 0), (0, _CDP - _CD)))
        .astype(jnp.bfloat16).reshape(_CHV, 128, 2), jnp.int32)
    wf = _ifog(jnp.pad(W_ih_f.T, ((0, _CDP - _CD), (0, 0)))).astype(jnp.bfloat16)
    wb = _ifog(jnp.pad(W_ih_b.T, ((0, _CDP - _CD), (0, 0)))).astype(jnp.bfloat16)
    wfe, wfo = wf[0::2], wf[1::2]
    wbe, wbo = wb[0::2], wb[1::2]
    uf = _ifog(W_hh_f.T).astype(jnp.bfloat16)
    ub = _ifog(W_hh_b.T).astype(jnp.bfloat16)
    bf = _ifog((b_ih_f + b_hh_f).reshape(1, _G))
    bb = _ifog((b_ih_b + b_hh_b).reshape(1, _G))
    lens_col = word_lens_encoder.reshape(_N, 1).astype(jnp.float32)

    wdom_p = jnp.pad(W_domain, ((0, 0), (0, _DDP - _DD)))
    xw, xe, ce, xd, db = _sc_gather()(words_p, ext_p, cidx, dom_p,
                                      domain_batch.astype(jnp.int32),
                                      W_word, W_ext, ctab, wdom_p)
    emb_t = ce.reshape(_LC, _N, 128)
    x2d = _tc_call(emb_t, lens_col, xw, xe, xd,
                   wfe, wfo, uf, bf, wbe, wbo, ub, bb)
    return x2d.reshape(_B, _S, _OUT), db[:, :_DD]


# speedup vs baseline: 7.0320x; 1.0536x over previous
import functools

import jax
import jax.numpy as jnp
from jax import lax
from jax.experimental import pallas as pl
from jax.experimental.pallas import tpu as pltpu
from jax.experimental.pallas import tpu_sc as plsc

_B, _S, _LC = 64, 50, 16
_CHV = 5000
_N = _B * _S
_WD, _CD, _CH, _DD = 128, 200, 64, 32
_CDP = 256
_DDP = 128
_G = 4 * _CH
_OUT = _WD + 2 * _CH + _DD

_NC, _NS = 2, 16
_NW = _NC * _NS
_NPAD = 4096
_WB = _NPAD // _NW
_CCH = 80
_CNCH = (_N * _LC) // (_NW * _CCH)
_NBUF = 6
_LEAD = 3

_BLK = 320
_NB = _N // _BLK


def _sc_body(words_ref, ext_ref, cidx_ref, dom_ref, domb_ref,
             wword_ref, wext_ref, ctab_ref, wdom_ref,
             xw_ref, xe_ref, ce_ref, xd_ref, db_ref,
             widx_v, eidx_v, didx_v, dbidx_v, cidx_v,
             wrows_v, erows_v, drows_v, dbrows_v, cbufs,
             wsem, esem, dsem, dbsem, gsems, osems):
    wid = lax.axis_index("s") * _NC + lax.axis_index("c")
    base = wid * _WB

    pltpu.sync_copy(words_ref.at[pl.ds(base, _WB)], widx_v)
    pltpu.sync_copy(ext_ref.at[pl.ds(base, _WB)], eidx_v)
    pltpu.sync_copy(dom_ref.at[pl.ds(base, _WB)], didx_v)
    pltpu.sync_copy(cidx_ref.at[wid], cidx_v)

    gw = pltpu.async_copy(wword_ref.at[widx_v], wrows_v, wsem)
    ge = pltpu.async_copy(wext_ref.at[eidx_v], erows_v, esem)
    gd = pltpu.async_copy(wdom_ref.at[didx_v], drows_v, dsem)

    @pl.when(wid < 8)
    def _():
        pltpu.sync_copy(domb_ref.at[pl.ds(wid * 8, 8)], dbidx_v)

    @pl.when(wid < 8)
    def _():
        pltpu.async_copy(wdom_ref.at[dbidx_v], dbrows_v, dbsem).wait()
        pltpu.sync_copy(dbrows_v, db_ref.at[pl.ds(wid * 8, 8)])

    cbase = wid * (_CNCH * _CCH)
    cg = [None] * (_CNCH + _LEAD)
    cw = [None] * _CNCH
    for k in range(_LEAD):
        cg[k] = pltpu.async_copy(ctab_ref.at[cidx_v.at[k]], cbufs[k % _NBUF],
                                 gsems[k % _NBUF])
    for k in range(_CNCH):
        b = k % _NBUF
        if k + _LEAD - _NBUF >= 0:
            cw[k + _LEAD - _NBUF].wait()
        if k + _LEAD < _CNCH:
            bb = (k + _LEAD) % _NBUF
            cg[k + _LEAD] = pltpu.async_copy(
                ctab_ref.at[cidx_v.at[k + _LEAD]], cbufs[bb], gsems[bb])
        cg[k].wait()
        cw[k] = pltpu.async_copy(
            cbufs[b], ce_ref.at[pl.ds(cbase + k * _CCH, _CCH)], osems[b])
    for k in range(max(0, _CNCH + _LEAD - _NBUF), _CNCH):
        cw[k].wait()

    gw.wait()
    ww = pltpu.async_copy(wrows_v, xw_ref.at[pl.ds(base, _WB)], wsem)
    ge.wait()
    we = pltpu.async_copy(erows_v, xe_ref.at[pl.ds(base, _WB)], esem)
    gd.wait()
    wd = pltpu.async_copy(drows_v, xd_ref.at[pl.ds(base, _WB)], dsem)
    ww.wait()
    we.wait()
    wd.wait()


@functools.cache
def _sc_gather():
    return pl.kernel(
        _sc_body,
        out_type=(
            jax.ShapeDtypeStruct((_NPAD, _WD), jnp.float32),
            jax.ShapeDtypeStruct((_NPAD, _WD), jnp.float32),
            jax.ShapeDtypeStruct((_N * _LC, 128), jnp.int32),
            jax.ShapeDtypeStruct((_NPAD, _DDP), jnp.float32),
            jax.ShapeDtypeStruct((_B, _DDP), jnp.float32),
        ),
        mesh=plsc.VectorSubcoreMesh(core_axis_name="c", subcore_axis_name="s",
                                    num_cores=_NC, num_subcores=_NS),
        scratch_types=[
            pltpu.VMEM((_WB,), jnp.int32),
            pltpu.VMEM((_WB,), jnp.int32),
            pltpu.VMEM((_WB,), jnp.int32),
            pltpu.VMEM((8,), jnp.int32),
            pltpu.VMEM((_CNCH, _CCH), jnp.int32),
            pltpu.VMEM((_WB, _WD), jnp.float32),
            pltpu.VMEM((_WB, _WD), jnp.float32),
            pltpu.VMEM((_WB, _DDP), jnp.float32),
            pltpu.VMEM((8, _DDP), jnp.float32),
            [pltpu.VMEM((_CCH, 128), jnp.int32) for _ in range(_NBUF)],
            pltpu.SemaphoreType.DMA,
            pltpu.SemaphoreType.DMA,
            pltpu.SemaphoreType.DMA,
            pltpu.SemaphoreType.DMA,
            [pltpu.SemaphoreType.DMA for _ in range(_NBUF)],
            [pltpu.SemaphoreType.DMA for _ in range(_NBUF)],
        ],
    )


def _pack_body(tab_ref, out_ref):
    x = tab_ref[...]
    lo = x[:, 0:128].astype(jnp.bfloat16)
    hi = jnp.concatenate(
        [x[:, 128:_CD], jnp.zeros((x.shape[0], _CDP - _CD), jnp.float32)],
        axis=1).astype(jnp.bfloat16)
    l16 = jax.lax.bitcast_convert_type(lo, jnp.uint16).astype(jnp.int32)
    h16 = jax.lax.bitcast_convert_type(hi, jnp.uint16).astype(jnp.int32)
    out_ref[...] = (h16 << 16) | l16


def _pack_table(char_table):
    return pl.pallas_call(
        _pack_body,
        grid=(5,),
        in_specs=[pl.BlockSpec((_CHV // 5, _CD), lambda i: (i, 0))],
        out_specs=pl.BlockSpec((_CHV // 5, 128), lambda i: (i, 0)),
        out_shape=jax.ShapeDtypeStruct((_CHV, 128), jnp.int32),
    )(char_table)


def _lstm_body(emb_ref, lens_ref, xw_ref, xe_ref, xd_ref,
               wf_ref, uf_ref, bf_ref, wb_ref, ub_ref, bb_ref,
               out_ref, pf_ref, pb_ref):
    v = emb_ref[...].reshape(_LC * _BLK, 128)
    e = jax.lax.bitcast_convert_type(v << 16, jnp.float32).astype(jnp.bfloat16)
    o = jax.lax.bitcast_convert_type(v & jnp.int32(-65536),
                                     jnp.float32).astype(jnp.bfloat16)
    pf_ref[...] = (jnp.dot(e, wf_ref[0:128], preferred_element_type=jnp.float32)
                   + jnp.dot(o, wf_ref[128:256], preferred_element_type=jnp.float32)
                   + bf_ref[...]).reshape(_LC, _BLK, _G)
    pb_ref[...] = (jnp.dot(e, wb_ref[0:128], preferred_element_type=jnp.float32)
                   + jnp.dot(o, wb_ref[128:256], preferred_element_type=jnp.float32)
                   + bb_ref[...]).reshape(_LC, _BLK, _G)
    lens = lens_ref[...]

    def run(p_ref, u_ref, times):
        u = u_ref[...]
        h = jnp.zeros((_BLK, _CH), jnp.float32)
        c = jnp.zeros((_BLK, _CH), jnp.float32)
        for t in times:
            g = p_ref[t] + jnp.dot(h.astype(jnp.bfloat16), u,
                                   preferred_element_type=jnp.float32)
            sg = jax.nn.sigmoid(g[:, 0:3 * _CH])
            gg = jnp.tanh(g[:, 3 * _CH:4 * _CH])
            i = sg[:, 0:_CH]
            f = sg[:, _CH:2 * _CH]
            o = sg[:, 2 * _CH:3 * _CH]
            c_new = f * c + i * gg
            h_new = o * jnp.tanh(c_new)
            m = lens > float(t)
            h = jnp.where(m, h_new, h)
            c = jnp.where(m, c_new, c)
        return h

    hf = run(pf_ref, uf_ref, range(_LC))
    hb = run(pb_ref, ub_ref, range(_LC - 1, -1, -1))
    out_ref[...] = jnp.concatenate(
        [xw_ref[...] + xe_ref[...], hf, hb, xd_ref[:, 0:_DD]], axis=1)


def _tc_call(emb_t, lens_col, xw, xe, xd, wf, uf, bf, wb, ub, bb):
    full = lambda s: pl.BlockSpec(s, lambda i: tuple(0 for _ in s))
    return pl.pallas_call(
        _lstm_body,
        grid=(_NB,),
        in_specs=[
            pl.BlockSpec((_LC, _BLK, 128), lambda i: (0, i, 0)),
            pl.BlockSpec((_BLK, 1), lambda i: (i, 0)),
            pl.BlockSpec((_BLK, _WD), lambda i: (i, 0)),
            pl.BlockSpec((_BLK, _WD), lambda i: (i, 0)),
            pl.BlockSpec((_BLK, _DDP), lambda i: (i, 0)),
            full((_CDP, _G)), full((_CH, _G)), full((1, _G)),
            full((_CDP, _G)), full((_CH, _G)), full((1, _G)),
        ],
        out_specs=pl.BlockSpec((_BLK, _OUT), lambda i: (i, 0)),
        out_shape=jax.ShapeDtypeStruct((_N, _OUT), jnp.float32),
        scratch_shapes=[
            pltpu.VMEM((_LC, _BLK, _G), jnp.float32),
            pltpu.VMEM((_LC, _BLK, _G), jnp.float32),
        ],
    )(emb_t, lens_col, xw, xe, xd, wf, uf, bf, wb, ub, bb)


def _ifog(w):
    return jnp.concatenate(
        [w[:, 0:2 * _CH], w[:, 3 * _CH:4 * _CH], w[:, 2 * _CH:3 * _CH]], axis=1)


def kernel(domain_batch, words, ext_words, tags, domains, word_lens_encoder,
           char_idxs_encoder, W_word, W_ext, W_domain, char_table,
           W_ih_f, W_hh_f, b_ih_f, b_hh_f, W_ih_b, W_hh_b, b_ih_b, b_hh_b):
    pad = _NPAD - _N
    words_p = jnp.pad(words.reshape(-1).astype(jnp.int32), (0, pad))
    ext_p = jnp.pad(ext_words.reshape(-1).astype(jnp.int32), (0, pad))
    dom_p = jnp.pad(domains.reshape(-1).astype(jnp.int32), (0, pad))
    cidx = char_idxs_encoder.astype(jnp.int32).T.reshape(_NW, _CNCH, _CCH)
    ctab = _pack_table(char_table)
    wf = _ifog(jnp.pad(W_ih_f.T, ((0, _CDP - _CD), (0, 0)))).astype(jnp.bfloat16)
    wb = _ifog(jnp.pad(W_ih_b.T, ((0, _CDP - _CD), (0, 0)))).astype(jnp.bfloat16)
    uf = _ifog(W_hh_f.T).astype(jnp.bfloat16)
    ub = _ifog(W_hh_b.T).astype(jnp.bfloat16)
    bf = _ifog((b_ih_f + b_hh_f).reshape(1, _G))
    bb = _ifog((b_ih_b + b_hh_b).reshape(1, _G))
    lens_col = word_lens_encoder.reshape(_N, 1).astype(jnp.float32)

    wdom_p = jnp.pad(W_domain, ((0, 0), (0, _DDP - _DD)))
    xw, xe, ce, xd, db = _sc_gather()(words_p, ext_p, cidx, dom_p,
                                      domain_batch.astype(jnp.int32),
                                      W_word, W_ext, ctab, wdom_p)
    emb_t = ce.reshape(_LC, _N, 128)
    x2d = _tc_call(emb_t, lens_col, xw, xe, xd,
                   wf, uf, bf, wb, ub, bb)
    return x2d.reshape(_B, _S, _OUT), db[:, :_DD]
